# Initial kernel scaffold; baseline (speedup 1.0000x reference)
import jax
import jax.numpy as jnp
from jax.experimental import pallas as pl


def kernel(x):
    raise NotImplementedError("write your pallas kernel here")


# speedup vs baseline: 5.6681x; 5.6681x over previous
import functools

import jax
import jax.numpy as jnp
from jax import lax
from jax.experimental import pallas as pl
from jax.experimental.pallas import tpu as pltpu
from jax.experimental.pallas import tpu_sc as plsc

B, C = 64, 16
IN_SIZE = 128 * 128
OUT_SIZE = 128 * 130
PAD = OUT_SIZE - IN_SIZE
ROWS = B * C
NL = 16
SEG = OUT_SIZE // NL
NCH_IN = IN_SIZE // NL
NCH_OUT = OUT_SIZE // NL
NC, NS = 2, 16
NW = NC * NS
ROWS_PER_W = ROWS // NW
NBITS = 8
RADIX = 1 << NBITS
INT_MIN = -2147483648


def _sc_body(xi_hbm, out_hbm, rev_hbm,
             key_a, key_b, val_a, val_b, hist, head_v, out_v, rev_v):
    cid = lax.axis_index("c")
    sid = lax.axis_index("s")
    wid = sid * NC + cid

    lane = jnp.arange(NL, dtype=jnp.int32)
    seg_base = lane * SEG
    ones = jnp.ones((NL,), jnp.int32)
    zeros = jnp.zeros((NL,), jnp.int32)

    pltpu.sync_copy(xi_hbm.at[0], head_v.at[pl.ds(0, IN_SIZE)])

    def _pad_head(j, c):
        head_v[pl.ds(IN_SIZE + NL * j, NL)] = zeros
        return c
    lax.fori_loop(0, PAD // NL, _pad_head, 0)

    def _row_body(r, carry):
        row = wid * ROWS_PER_W + r
        pltpu.sync_copy(xi_hbm.at[row], key_a.at[pl.ds(0, IN_SIZE)])

        def _xform(j, c):
            v = key_a[pl.ds(NL * j, NL)]
            s = lax.shift_right_arithmetic(v, 31)
            key_a[pl.ds(NL * j, NL)] = v ^ (s | jnp.int32(INT_MIN))
            return c
        lax.fori_loop(0, NCH_IN, _xform, 0)

        def _pad_keys(j, c):
            key_a[pl.ds(IN_SIZE + NL * j, NL)] = jnp.full((NL,), INT_MIN,
                                                          jnp.int32)
            return c
        lax.fori_loop(0, PAD // NL, _pad_keys, 0)

        for p in range(4):
            if p % 2 == 0:
                src_k, src_v, dst_k, dst_v = key_a, val_a, key_b, val_b
            else:
                src_k, src_v, dst_k, dst_v = key_b, val_b, key_a, val_a
            shift = NBITS * p

            def _zero_hist(j, c):
                hist[pl.ds(NL * j, NL)] = zeros
                return c
            lax.fori_loop(0, RADIX * NL // NL, _zero_hist, 0)

            def _hist_body(i, c):
                idx = seg_base + i
                ku = plsc.load_gather(src_k, [idx])
                d = lax.shift_right_logical(ku, shift) & (RADIX - 1)
                plsc.addupdate_scatter(hist, [d * NL + lane], ones)
                return c
            lax.fori_loop(0, SEG, _hist_body, 0)

            def _scan_body(j, run):
                v = hist[pl.ds(NL * j, NL)]
                inc = plsc.cumsum(v)
                hist[pl.ds(NL * j, NL)] = (run + inc) - v
                return run + jnp.sum(v)
            lax.fori_loop(0, RADIX, _scan_body, jnp.int32(0))

            def _perm_body(i, c):
                idx = seg_base + i
                ku = plsc.load_gather(src_k, [idx])
                d = lax.shift_right_logical(ku, shift) & (RADIX - 1)
                haddr = d * NL + lane
                dest = plsc.load_gather(hist, [haddr])
                plsc.store_scatter(hist, [haddr], dest + 1)
                plsc.store_scatter(dst_k, [dest], ku)
                if p == 0:
                    vv = idx
                else:
                    vv = plsc.load_gather(src_v, [idx])
                plsc.store_scatter(dst_v, [dest], vv)
                return c
            lax.fori_loop(0, SEG, _perm_body, 0)

        def _final_body(j, c):
            svals = val_a[pl.ds(NL * j, NL)]
            out_v[pl.ds(NL * j, NL)] = plsc.load_gather(head_v, [svals])
            plsc.store_scatter(rev_v, [svals], NL * j + lane)
            return c
        lax.fori_loop(0, NCH_OUT, _final_body, 0)

        pltpu.sync_copy(out_v, out_hbm.at[row])
        pltpu.sync_copy(rev_v, rev_hbm.at[row])
        return carry

    lax.fori_loop(0, ROWS_PER_W, _row_body, 0)


@jax.jit
def kernel(x):
    xr = jnp.reshape(x, (ROWS, IN_SIZE))
    xi = lax.bitcast_convert_type(xr, jnp.int32)

    sc = pl.kernel(
        _sc_body,
        out_type=[
            jax.ShapeDtypeStruct((ROWS, OUT_SIZE), jnp.int32),
            jax.ShapeDtypeStruct((ROWS, OUT_SIZE), jnp.int32),
        ],
        mesh=plsc.VectorSubcoreMesh(core_axis_name="c", subcore_axis_name="s"),
        compiler_params=pltpu.CompilerParams(needs_layout_passes=False),
        scratch_types=[
            pltpu.VMEM((OUT_SIZE,), jnp.int32),
            pltpu.VMEM((OUT_SIZE,), jnp.int32),
            pltpu.VMEM((OUT_SIZE,), jnp.int32),
            pltpu.VMEM((OUT_SIZE,), jnp.int32),
            pltpu.VMEM((RADIX * NL,), jnp.int32),
            pltpu.VMEM((OUT_SIZE,), jnp.int32),
            pltpu.VMEM((OUT_SIZE,), jnp.int32),
            pltpu.VMEM((OUT_SIZE,), jnp.int32),
        ],
    )
    out_bits, rev = sc(xi)
    out = lax.bitcast_convert_type(out_bits, jnp.float32)
    out = jnp.reshape(out, (B, C, 128, 130))
    rev = jnp.reshape(rev, (B, C, OUT_SIZE))
    return (out, rev)


# speedup vs baseline: 7.1925x; 1.2689x over previous
import functools

import jax
import jax.numpy as jnp
from jax import lax
from jax.experimental import pallas as pl
from jax.experimental.pallas import tpu as pltpu
from jax.experimental.pallas import tpu_sc as plsc

B, C = 64, 16
IN_SIZE = 128 * 128
OUT_SIZE = 128 * 130
PAD = OUT_SIZE - IN_SIZE
ROWS = B * C
NL = 16
NSTREAM = 4
VL = NL * NSTREAM
SEG = OUT_SIZE // VL
NCH_IN = IN_SIZE // NL
NCH_OUT = OUT_SIZE // NL
NC, NS = 2, 16
NW = NC * NS
ROWS_PER_W = ROWS // NW
NBITS = 8
RADIX = 1 << NBITS
HSIZE = RADIX * VL
INT_MIN = -2147483648


def _sc_body(xi_hbm, out_hbm, rev_hbm,
             key_a, key_b, val_a, val_b, hist, head_v):
    cid = lax.axis_index("c")
    sid = lax.axis_index("s")
    wid = sid * NC + cid

    lane = jnp.arange(NL, dtype=jnp.int32)
    seg_base = lane * SEG
    ones = jnp.ones((NL,), jnp.int32)
    zeros = jnp.zeros((NL,), jnp.int32)

    pltpu.sync_copy(xi_hbm.at[0], head_v.at[pl.ds(0, IN_SIZE)])

    def _pad_head(j, c):
        head_v[pl.ds(IN_SIZE + NL * j, NL)] = zeros
        return c
    lax.fori_loop(0, PAD // NL, _pad_head, 0)

    def _row_body(r, carry):
        row = wid * ROWS_PER_W + r
        pltpu.sync_copy(xi_hbm.at[row], key_a.at[pl.ds(0, IN_SIZE)])

        def _xform(j, c):
            v = key_a[pl.ds(NL * j, NL)]
            s = lax.shift_right_arithmetic(v, 31)
            key_a[pl.ds(NL * j, NL)] = v ^ (s | jnp.int32(INT_MIN))
            return c
        lax.fori_loop(0, NCH_IN, _xform, 0, unroll=4)

        def _pad_keys(j, c):
            key_a[pl.ds(IN_SIZE + NL * j, NL)] = jnp.full((NL,), INT_MIN,
                                                          jnp.int32)
            return c
        lax.fori_loop(0, PAD // NL, _pad_keys, 0)

        for p in range(4):
            if p % 2 == 0:
                src_k, src_v, dst_k, dst_v = key_a, val_a, key_b, val_b
            else:
                src_k, src_v, dst_k, dst_v = key_b, val_b, key_a, val_a
            shift = NBITS * p

            def _zero_hist(j, c):
                hist[pl.ds(NL * j, NL)] = zeros
                return c
            lax.fori_loop(0, HSIZE // NL, _zero_hist, 0, unroll=8)

            def _hist_body(i, c):
                for s in range(NSTREAM):
                    idx = seg_base + (i + (NL * SEG) * s)
                    ku = plsc.load_gather(src_k, [idx])
                    d = lax.shift_right_logical(ku, shift) & (RADIX - 1)
                    haddr = d * VL + (lane + NL * s)
                    plsc.addupdate_scatter(hist, [haddr], ones)
                return c
            lax.fori_loop(0, SEG, _hist_body, 0, unroll=2)

            def _scan_body(j, run):
                v = hist[pl.ds(NL * j, NL)]
                inc = plsc.cumsum(v)
                hist[pl.ds(NL * j, NL)] = (run + inc) - v
                return run + jnp.sum(v)
            lax.fori_loop(0, HSIZE // NL, _scan_body, jnp.int32(0), unroll=4)

            def _perm_body(i, c):
                for s in range(NSTREAM):
                    idx = seg_base + (i + (NL * SEG) * s)
                    ku = plsc.load_gather(src_k, [idx])
                    d = lax.shift_right_logical(ku, shift) & (RADIX - 1)
                    haddr = d * VL + (lane + NL * s)
                    dest = plsc.load_gather(hist, [haddr])
                    plsc.store_scatter(hist, [haddr], dest + 1)
                    plsc.store_scatter(dst_k, [dest], ku)
                    if p == 0:
                        vv = idx
                    else:
                        vv = plsc.load_gather(src_v, [idx])
                    plsc.store_scatter(dst_v, [dest], vv)
                return c
            lax.fori_loop(0, SEG, _perm_body, 0, unroll=2)

        def _final_body(j, c):
            svals = val_a[pl.ds(NL * j, NL)]
            key_b[pl.ds(NL * j, NL)] = plsc.load_gather(head_v, [svals])
            plsc.store_scatter(val_b, [svals], NL * j + lane)
            return c
        lax.fori_loop(0, NCH_OUT, _final_body, 0, unroll=2)

        pltpu.sync_copy(key_b, out_hbm.at[row])
        pltpu.sync_copy(val_b, rev_hbm.at[row])
        return carry

    lax.fori_loop(0, ROWS_PER_W, _row_body, 0)


@jax.jit
def kernel(x):
    xr = jnp.reshape(x, (ROWS, IN_SIZE))
    xi = lax.bitcast_convert_type(xr, jnp.int32)

    sc = pl.kernel(
        _sc_body,
        out_type=[
            jax.ShapeDtypeStruct((ROWS, OUT_SIZE), jnp.int32),
            jax.ShapeDtypeStruct((ROWS, OUT_SIZE), jnp.int32),
        ],
        mesh=plsc.VectorSubcoreMesh(core_axis_name="c", subcore_axis_name="s"),
        compiler_params=pltpu.CompilerParams(needs_layout_passes=False),
        scratch_types=[
            pltpu.VMEM((OUT_SIZE,), jnp.int32),
            pltpu.VMEM((OUT_SIZE,), jnp.int32),
            pltpu.VMEM((OUT_SIZE,), jnp.int32),
            pltpu.VMEM((OUT_SIZE,), jnp.int32),
            pltpu.VMEM((HSIZE,), jnp.int32),
            pltpu.VMEM((OUT_SIZE,), jnp.int32),
        ],
    )
    out_bits, rev = sc(xi)
    out = lax.bitcast_convert_type(out_bits, jnp.float32)
    out = jnp.reshape(out, (B, C, 128, 130))
    rev = jnp.reshape(rev, (B, C, OUT_SIZE))
    return (out, rev)


# speedup vs baseline: 10.7968x; 1.5011x over previous
import functools

import jax
import jax.numpy as jnp
from jax import lax
from jax.experimental import pallas as pl
from jax.experimental.pallas import tpu as pltpu
from jax.experimental.pallas import tpu_sc as plsc

B, C = 64, 16
IN_SIZE = 128 * 128
OUT_SIZE = 128 * 130
PAD = OUT_SIZE - IN_SIZE
ROWS = B * C
NL = 16
NSTREAM = 4
VL = NL * NSTREAM
SEG = OUT_SIZE // VL
NCH_IN = IN_SIZE // NL
NCH_OUT = OUT_SIZE // NL
NC, NS = 2, 16
NW = NC * NS
ROWS_PER_W = ROWS // NW
NBITS = 8
RADIX = 1 << NBITS
HSIZE = RADIX * VL
INT_MIN = -2147483648


def _sc_body(xi_hbm, out_hbm, rev_hbm,
             key_a, key_b, val_a, val_b, hist, ctr, head_v):
    cid = lax.axis_index("c")
    sid = lax.axis_index("s")
    wid = sid * NC + cid

    lane = jnp.arange(NL, dtype=jnp.int32)
    seg_base = lane * SEG
    ones = jnp.ones((NL,), jnp.int32)
    zeros = jnp.zeros((NL,), jnp.int32)

    pltpu.sync_copy(xi_hbm.at[0], head_v.at[pl.ds(0, IN_SIZE)])

    def _pad_head(j, c):
        head_v[pl.ds(IN_SIZE + NL * j, NL)] = zeros
        return c
    lax.fori_loop(0, PAD // NL, _pad_head, 0)

    def _zero_hist(j, c):
        hist[pl.ds(NL * j, NL)] = zeros
        return c
    lax.fori_loop(0, HSIZE // NL, _zero_hist, 0, unroll=4)

    def _row_body(r, carry):
        row = wid * ROWS_PER_W + r
        pltpu.sync_copy(xi_hbm.at[row], key_a.at[pl.ds(0, IN_SIZE)])

        def _xform(j, c):
            v = key_a[pl.ds(NL * j, NL)]
            s = lax.shift_right_arithmetic(v, 31)
            key_a[pl.ds(NL * j, NL)] = v ^ (s | jnp.int32(INT_MIN))
            return c
        lax.fori_loop(0, NCH_IN, _xform, 0, unroll=4)

        def _pad_keys(j, c):
            key_a[pl.ds(IN_SIZE + NL * j, NL)] = jnp.full((NL,), INT_MIN,
                                                          jnp.int32)
            return c
        lax.fori_loop(0, PAD // NL, _pad_keys, 0)

        for p in range(4):
            if p % 2 == 0:
                src_k, src_v, dst_k, dst_v = key_a, val_a, key_b, val_b
            else:
                src_k, src_v, dst_k, dst_v = key_b, val_b, key_a, val_a
            shift = NBITS * p

            def _hist_body(i, c):
                kus = [plsc.load_gather(src_k,
                                        [seg_base + (i + (NL * SEG) * s)])
                       for s in range(NSTREAM)]
                hads = []
                for s in range(NSTREAM):
                    d = lax.shift_right_logical(kus[s], shift) & (RADIX - 1)
                    hads.append(d * VL + (lane + NL * s))
                for s in range(NSTREAM):
                    plsc.addupdate_scatter(hist, [hads[s]], ones)
                return c
            lax.fori_loop(0, SEG, _hist_body, 0)

            def _scan_body(j, run):
                v = hist[pl.ds(NL * j, NL)]
                inc = plsc.cumsum(v)
                ctr[pl.ds(NL * j, NL)] = (run + inc) - v
                hist[pl.ds(NL * j, NL)] = zeros
                return run + jnp.sum(v)
            lax.fori_loop(0, HSIZE // NL, _scan_body, jnp.int32(0),
                          unroll=2)

            def _perm_body(i, c):
                idxs = [seg_base + (i + (NL * SEG) * s)
                        for s in range(NSTREAM)]
                kus = [plsc.load_gather(src_k, [idxs[s]])
                       for s in range(NSTREAM)]
                if p == 0:
                    vvs = idxs
                else:
                    vvs = [plsc.load_gather(src_v, [idxs[s]])
                           for s in range(NSTREAM)]
                hads = []
                for s in range(NSTREAM):
                    d = lax.shift_right_logical(kus[s], shift) & (RADIX - 1)
                    hads.append(d * VL + (lane + NL * s))
                dests = [plsc.load_gather(ctr, [hads[s]])
                         for s in range(NSTREAM)]
                for s in range(NSTREAM):
                    plsc.store_scatter(ctr, [hads[s]], dests[s] + 1)
                    plsc.store_scatter(dst_k, [dests[s]], kus[s])
                    plsc.store_scatter(dst_v, [dests[s]], vvs[s])
                return c
            lax.fori_loop(0, SEG, _perm_body, 0)

        def _final_body(j, c):
            svs = [val_a[pl.ds(NL * (NSTREAM * j + q), NL)]
                   for q in range(NSTREAM)]
            hvs = [plsc.load_gather(head_v, [svs[q]])
                   for q in range(NSTREAM)]
            for q in range(NSTREAM):
                key_b[pl.ds(NL * (NSTREAM * j + q), NL)] = hvs[q]
                plsc.store_scatter(val_b, [svs[q]],
                                   NL * (NSTREAM * j + q) + lane)
            return c
        lax.fori_loop(0, NCH_OUT // NSTREAM, _final_body, 0)

        pltpu.sync_copy(key_b, out_hbm.at[row])
        pltpu.sync_copy(val_b, rev_hbm.at[row])
        return carry

    lax.fori_loop(0, ROWS_PER_W, _row_body, 0)


@jax.jit
def kernel(x):
    xr = jnp.reshape(x, (ROWS, IN_SIZE))
    xi = lax.bitcast_convert_type(xr, jnp.int32)

    sc = pl.kernel(
        _sc_body,
        out_type=[
            jax.ShapeDtypeStruct((ROWS, OUT_SIZE), jnp.int32),
            jax.ShapeDtypeStruct((ROWS, OUT_SIZE), jnp.int32),
        ],
        mesh=plsc.VectorSubcoreMesh(core_axis_name="c", subcore_axis_name="s"),
        compiler_params=pltpu.CompilerParams(needs_layout_passes=False),
        scratch_types=[
            pltpu.VMEM((OUT_SIZE,), jnp.int32),
            pltpu.VMEM((OUT_SIZE,), jnp.int32),
            pltpu.VMEM((OUT_SIZE,), jnp.int32),
            pltpu.VMEM((OUT_SIZE,), jnp.int32),
            pltpu.VMEM((HSIZE,), jnp.int32),
            pltpu.VMEM((HSIZE,), jnp.int32),
            pltpu.VMEM((OUT_SIZE,), jnp.int32),
        ],
    )
    out_bits, rev = sc(xi)
    out = lax.bitcast_convert_type(out_bits, jnp.float32)
    out = jnp.reshape(out, (B, C, 128, 130))
    rev = jnp.reshape(rev, (B, C, OUT_SIZE))
    return (out, rev)


# speedup vs baseline: 13.7966x; 1.2778x over previous
import functools

import jax
import jax.numpy as jnp
from jax import lax
from jax.experimental import pallas as pl
from jax.experimental.pallas import tpu as pltpu
from jax.experimental.pallas import tpu_sc as plsc

B, C = 64, 16
IN_SIZE = 128 * 128
OUT_SIZE = 128 * 130
PAD = OUT_SIZE - IN_SIZE
ROWS = B * C
NL = 16
NSTREAM = 4
VL = NL * NSTREAM
SEG = OUT_SIZE // VL
NCH_IN = IN_SIZE // NL
NCH_OUT = OUT_SIZE // NL
NC, NS = 2, 16
NW = NC * NS
ROWS_PER_W = ROWS // NW
NBITS = 8
RADIX = 1 << NBITS
HSIZE = RADIX * VL
INT_MIN = -2147483648


def _sc_body(xi_hbm, out_hbm, rev_hbm,
             key_a, key_b, val_a, val_b, hist, ctr, aux, head_v):
    cid = lax.axis_index("c")
    sid = lax.axis_index("s")
    wid = sid * NC + cid

    lane = jnp.arange(NL, dtype=jnp.int32)
    seg_base = lane * SEG
    ones = jnp.ones((NL,), jnp.int32)
    zeros = jnp.zeros((NL,), jnp.int32)

    pltpu.sync_copy(xi_hbm.at[0], head_v.at[pl.ds(0, IN_SIZE)])

    def _pad_head(j, c):
        head_v[pl.ds(IN_SIZE + NL * j, NL)] = zeros
        return c
    lax.fori_loop(0, PAD // NL, _pad_head, 0)

    def _zero_hist(j, c):
        hist[pl.ds(NL * j, NL)] = zeros
        return c
    lax.fori_loop(0, HSIZE // NL, _zero_hist, 0, unroll=4)

    def _row_body(r, carry):
        row = wid * ROWS_PER_W + r
        pltpu.sync_copy(xi_hbm.at[row], key_a.at[pl.ds(0, IN_SIZE)])

        def _xform(j, c):
            v = key_a[pl.ds(NL * j, NL)]
            s = lax.shift_right_arithmetic(v, 31)
            key_a[pl.ds(NL * j, NL)] = v ^ (s | jnp.int32(INT_MIN))
            return c
        lax.fori_loop(0, NCH_IN, _xform, 0, unroll=4)

        def _pad_keys(j, c):
            key_a[pl.ds(IN_SIZE + NL * j, NL)] = jnp.full((NL,), INT_MIN,
                                                          jnp.int32)
            return c
        lax.fori_loop(0, PAD // NL, _pad_keys, 0)

        for p in range(4):
            if p % 2 == 0:
                src_k, src_v, dst_k, dst_v = key_a, val_a, key_b, val_b
            else:
                src_k, src_v, dst_k, dst_v = key_b, val_b, key_a, val_a
            shift = NBITS * p

            def _hist_body(i, c):
                kus = [plsc.load_gather(src_k,
                                        [seg_base + (i + (NL * SEG) * s)])
                       for s in range(NSTREAM)]
                hads = []
                for s in range(NSTREAM):
                    d = lax.shift_right_logical(kus[s], shift) & (RADIX - 1)
                    hads.append(d * VL + (lane + NL * s))
                for s in range(NSTREAM):
                    plsc.addupdate_scatter(hist, [hads[s]], ones)
                return c
            lax.fori_loop(0, SEG, _hist_body, 0)

            def _b1_body(j, c):
                vs = [hist[pl.ds(NL * (4 * j + q), NL)] for q in range(4)]
                cs = [plsc.cumsum(v) for v in vs]
                for q in range(4):
                    hist[pl.ds(NL * (4 * j + q), NL)] = cs[q]
                return c
            lax.fori_loop(0, HSIZE // NL // 4, _b1_body, 0)

            def _b2_body(k, run):
                t = plsc.load_gather(
                    hist, [(NL * k + lane) * NL + (NL - 1)])
                inc = plsc.cumsum(t)
                aux[pl.ds(NL * k, NL)] = (run + inc) - t
                return run + jnp.sum(t)
            lax.fori_loop(0, HSIZE // NL // NL, _b2_body, jnp.int32(0))

            lm1 = jnp.maximum(lane - 1, 0)
            def _b3_body(j, c):
                av = aux[pl.ds(4 * j, NL)]
                bases = [av[q] for q in range(4)]
                shs = [plsc.load_gather(hist,
                                        [NL * (4 * j + q) + lm1])
                       for q in range(4)]
                for q in range(4):
                    excl = jnp.where(lane > 0, shs[q], 0) + bases[q]
                    ctr[pl.ds(NL * (4 * j + q), NL)] = excl
                for q in range(4):
                    hist[pl.ds(NL * (4 * j + q), NL)] = zeros
                return c
            lax.fori_loop(0, HSIZE // NL // 4, _b3_body, 0)

            def _perm_body(i, c):
                idxs = [seg_base + (i + (NL * SEG) * s)
                        for s in range(NSTREAM)]
                kus = [plsc.load_gather(src_k, [idxs[s]])
                       for s in range(NSTREAM)]
                if p == 0:
                    vvs = idxs
                else:
                    vvs = [plsc.load_gather(src_v, [idxs[s]])
                           for s in range(NSTREAM)]
                hads = []
                for s in range(NSTREAM):
                    d = lax.shift_right_logical(kus[s], shift) & (RADIX - 1)
                    hads.append(d * VL + (lane + NL * s))
                dests = [plsc.load_gather(ctr, [hads[s]])
                         for s in range(NSTREAM)]
                for s in range(NSTREAM):
                    plsc.store_scatter(ctr, [hads[s]], dests[s] + 1)
                    plsc.store_scatter(dst_k, [dests[s]], kus[s])
                    plsc.store_scatter(dst_v, [dests[s]], vvs[s])
                return c
            lax.fori_loop(0, SEG, _perm_body, 0)

        def _final_body(j, c):
            svs = [val_a[pl.ds(NL * (NSTREAM * j + q), NL)]
                   for q in range(NSTREAM)]
            hvs = [plsc.load_gather(head_v, [svs[q]])
                   for q in range(NSTREAM)]
            for q in range(NSTREAM):
                key_b[pl.ds(NL * (NSTREAM * j + q), NL)] = hvs[q]
                plsc.store_scatter(val_b, [svs[q]],
                                   NL * (NSTREAM * j + q) + lane)
            return c
        lax.fori_loop(0, NCH_OUT // NSTREAM, _final_body, 0)

        pltpu.sync_copy(key_b, out_hbm.at[row])
        pltpu.sync_copy(val_b, rev_hbm.at[row])
        return carry

    lax.fori_loop(0, ROWS_PER_W, _row_body, 0)


@jax.jit
def kernel(x):
    xr = jnp.reshape(x, (ROWS, IN_SIZE))
    xi = lax.bitcast_convert_type(xr, jnp.int32)

    sc = pl.kernel(
        _sc_body,
        out_type=[
            jax.ShapeDtypeStruct((ROWS, OUT_SIZE), jnp.int32),
            jax.ShapeDtypeStruct((ROWS, OUT_SIZE), jnp.int32),
        ],
        mesh=plsc.VectorSubcoreMesh(core_axis_name="c", subcore_axis_name="s"),
        compiler_params=pltpu.CompilerParams(needs_layout_passes=False),
        scratch_types=[
            pltpu.VMEM((OUT_SIZE,), jnp.int32),
            pltpu.VMEM((OUT_SIZE,), jnp.int32),
            pltpu.VMEM((OUT_SIZE,), jnp.int32),
            pltpu.VMEM((OUT_SIZE,), jnp.int32),
            pltpu.VMEM((HSIZE,), jnp.int32),
            pltpu.VMEM((HSIZE,), jnp.int32),
            pltpu.VMEM((HSIZE // NL + NL,), jnp.int32),
            pltpu.VMEM((OUT_SIZE,), jnp.int32),
        ],
    )
    out_bits, rev = sc(xi)
    out = lax.bitcast_convert_type(out_bits, jnp.float32)
    out = jnp.reshape(out, (B, C, 128, 130))
    rev = jnp.reshape(rev, (B, C, OUT_SIZE))
    return (out, rev)


# speedup vs baseline: 16.1079x; 1.1675x over previous
import functools

import jax
import jax.numpy as jnp
from jax import lax
from jax.experimental import pallas as pl
from jax.experimental.pallas import tpu as pltpu
from jax.experimental.pallas import tpu_sc as plsc

B, C = 64, 16
IN_SIZE = 128 * 128
OUT_SIZE = 128 * 130
PAD = OUT_SIZE - IN_SIZE
ROWS = B * C
NL = 16
NSTREAM = 4
VL = NL * NSTREAM
SEG = OUT_SIZE // VL
NCH_IN = IN_SIZE // NL
NCH_OUT = OUT_SIZE // NL
NC, NS = 2, 16
NW = NC * NS
ROWS_PER_W = ROWS // NW
NBITS = 8
RADIX = 1 << NBITS
HSIZE = RADIX * VL
INT_MIN = -2147483648


def _sc_body(xi_hbm, out_hbm, rev_hbm,
             key_a, key_b, val_a, val_b, hist, ctr, aux, head_v):
    cid = lax.axis_index("c")
    sid = lax.axis_index("s")
    wid = sid * NC + cid

    lane = jnp.arange(NL, dtype=jnp.int32)
    seg_base = lane * SEG
    ones = jnp.ones((NL,), jnp.int32)
    zeros = jnp.zeros((NL,), jnp.int32)

    pltpu.sync_copy(xi_hbm.at[0], head_v.at[pl.ds(0, IN_SIZE)])

    def _pad_head(j, c):
        head_v[pl.ds(IN_SIZE + NL * j, NL)] = zeros
        return c
    lax.fori_loop(0, PAD // NL, _pad_head, 0)

    def _zero_hist(j, c):
        hist[pl.ds(NL * j, NL)] = zeros
        return c
    lax.fori_loop(0, HSIZE // NL, _zero_hist, 0, unroll=4)

    def _digit(ku, shift):
        t = ku ^ (lax.shift_right_arithmetic(ku, 31) | jnp.int32(INT_MIN))
        return lax.shift_right_logical(t, shift) & (RADIX - 1)

    def _row_body(r, carry):
        row = wid * ROWS_PER_W + r
        pltpu.sync_copy(xi_hbm.at[row], key_a.at[pl.ds(0, IN_SIZE)])

        def _pad_keys(j, c):
            key_a[pl.ds(IN_SIZE + NL * j, NL)] = zeros
            return c
        lax.fori_loop(0, PAD // NL, _pad_keys, 0)

        for p in range(4):
            if p % 2 == 0:
                src_k, src_v, dst_k, dst_v = key_a, val_a, key_b, val_b
            else:
                src_k, src_v, dst_k, dst_v = key_b, val_b, key_a, val_a
            shift = NBITS * p

            def _a_load(i):
                hads = []
                for s in range(NSTREAM):
                    ku = plsc.load_gather(src_k,
                                          [seg_base + (i + (NL * SEG) * s)])
                    hads.append(_digit(ku, shift) * VL + (lane + NL * s))
                return tuple(hads)

            def _hist_body(i, hads):
                nxt = _a_load(jnp.minimum(i + 1, SEG - 1))
                for s in range(NSTREAM):
                    plsc.addupdate_scatter(hist, [hads[s]], ones)
                return nxt
            lax.fori_loop(0, SEG, _hist_body, _a_load(0))

            def _b1_body(j, c):
                vs = [hist[pl.ds(NL * (4 * j + q), NL)] for q in range(4)]
                cs = [plsc.cumsum(v) for v in vs]
                for q in range(4):
                    hist[pl.ds(NL * (4 * j + q), NL)] = cs[q]
                return c
            lax.fori_loop(0, HSIZE // NL // 4, _b1_body, 0)

            def _b2_body(k, run):
                t = plsc.load_gather(
                    hist, [(NL * k + lane) * NL + (NL - 1)])
                inc = plsc.cumsum(t)
                aux[pl.ds(NL * k, NL)] = (run + inc) - t
                return run + jnp.sum(t)
            lax.fori_loop(0, HSIZE // NL // NL, _b2_body, jnp.int32(0))

            lm1 = jnp.maximum(lane - 1, 0)
            def _b3_body(j, c):
                av = aux[pl.ds(4 * j, NL)]
                bases = [av[q] for q in range(4)]
                shs = [plsc.load_gather(hist,
                                        [NL * (4 * j + q) + lm1])
                       for q in range(4)]
                for q in range(4):
                    excl = jnp.where(lane > 0, shs[q], 0) + bases[q]
                    ctr[pl.ds(NL * (4 * j + q), NL)] = excl
                for q in range(4):
                    hist[pl.ds(NL * (4 * j + q), NL)] = zeros
                return c
            lax.fori_loop(0, HSIZE // NL // 4, _b3_body, 0)

            def _c_load(i):
                idxs = [seg_base + (i + (NL * SEG) * s)
                        for s in range(NSTREAM)]
                kus = [plsc.load_gather(src_k, [idxs[s]])
                       for s in range(NSTREAM)]
                if p == 0:
                    vvs = idxs
                else:
                    vvs = [plsc.load_gather(src_v, [idxs[s]])
                           for s in range(NSTREAM)]
                hads = [_digit(kus[s], shift) * VL + (lane + NL * s)
                        for s in range(NSTREAM)]
                return tuple(kus), tuple(vvs), tuple(hads)

            def _perm_body(i, st):
                kus, vvs, hads = st
                nxt = _c_load(jnp.minimum(i + 1, SEG - 1))
                dests = [plsc.load_gather(ctr, [hads[s]])
                         for s in range(NSTREAM)]
                for s in range(NSTREAM):
                    plsc.store_scatter(ctr, [hads[s]], dests[s] + 1)
                    plsc.store_scatter(dst_k, [dests[s]], kus[s])
                    plsc.store_scatter(dst_v, [dests[s]], vvs[s])
                return nxt
            lax.fori_loop(0, SEG, _perm_body, _c_load(0))

        def _f_load(j):
            svs = [val_a[pl.ds(NL * (NSTREAM * j + q), NL)]
                   for q in range(NSTREAM)]
            hvs = [plsc.load_gather(head_v, [svs[q]])
                   for q in range(NSTREAM)]
            return tuple(svs), tuple(hvs)

        def _final_body(j, st):
            svs, hvs = st
            nxt = _f_load(jnp.minimum(j + 1, NCH_OUT // NSTREAM - 1))
            for q in range(NSTREAM):
                key_b[pl.ds(NL * (NSTREAM * j + q), NL)] = hvs[q]
                plsc.store_scatter(val_b, [svs[q]],
                                   NL * (NSTREAM * j + q) + lane)
            return nxt
        lax.fori_loop(0, NCH_OUT // NSTREAM, _final_body, _f_load(0))

        pltpu.sync_copy(key_b, out_hbm.at[row])
        pltpu.sync_copy(val_b, rev_hbm.at[row])
        return carry

    lax.fori_loop(0, ROWS_PER_W, _row_body, 0)


@jax.jit
def kernel(x):
    xr = jnp.reshape(x, (ROWS, IN_SIZE))
    xi = lax.bitcast_convert_type(xr, jnp.int32)

    sc = pl.kernel(
        _sc_body,
        out_type=[
            jax.ShapeDtypeStruct((ROWS, OUT_SIZE), jnp.int32),
            jax.ShapeDtypeStruct((ROWS, OUT_SIZE), jnp.int32),
        ],
        mesh=plsc.VectorSubcoreMesh(core_axis_name="c", subcore_axis_name="s"),
        compiler_params=pltpu.CompilerParams(needs_layout_passes=False),
        scratch_types=[
            pltpu.VMEM((OUT_SIZE,), jnp.int32),
            pltpu.VMEM((OUT_SIZE,), jnp.int32),
            pltpu.VMEM((OUT_SIZE,), jnp.int32),
            pltpu.VMEM((OUT_SIZE,), jnp.int32),
            pltpu.VMEM((HSIZE,), jnp.int32),
            pltpu.VMEM((HSIZE,), jnp.int32),
            pltpu.VMEM((HSIZE // NL + NL,), jnp.int32),
            pltpu.VMEM((OUT_SIZE,), jnp.int32),
        ],
    )
    out_bits, rev = sc(xi)
    out = lax.bitcast_convert_type(out_bits, jnp.float32)
    out = jnp.reshape(out, (B, C, 128, 130))
    rev = jnp.reshape(rev, (B, C, OUT_SIZE))
    return (out, rev)


# speedup vs baseline: 16.6362x; 1.0328x over previous
import functools

import jax
import jax.numpy as jnp
from jax import lax
from jax.experimental import pallas as pl
from jax.experimental.pallas import tpu as pltpu
from jax.experimental.pallas import tpu_sc as plsc

B, C = 64, 16
IN_SIZE = 128 * 128
OUT_SIZE = 128 * 130
PAD = OUT_SIZE - IN_SIZE
ROWS = B * C
NL = 16
NSTREAM = 4
VL = NL * NSTREAM
SEG = 261
LAST_LEN = OUT_SIZE - (VL - 1) * SEG
NCH_IN = IN_SIZE // NL
NCH_OUT = OUT_SIZE // NL
NC, NS = 2, 16
NW = NC * NS
ROWS_PER_W = ROWS // NW
NBITS = 8
RADIX = 1 << NBITS
HSIZE = RADIX * VL
INT_MIN = -2147483648


def _sc_body(xi_hbm, out_hbm, rev_hbm,
             key_a, key_b, val_a, val_b, hist, ctr, aux, tot, head_v):
    cid = lax.axis_index("c")
    sid = lax.axis_index("s")
    wid = sid * NC + cid

    lane = jnp.arange(NL, dtype=jnp.int32)
    seg_base = lane * SEG
    ones = jnp.ones((NL,), jnp.int32)
    zeros = jnp.zeros((NL,), jnp.int32)

    pltpu.sync_copy(xi_hbm.at[0], head_v.at[pl.ds(0, IN_SIZE)])

    def _pad_head(j, c):
        head_v[pl.ds(IN_SIZE + NL * j, NL)] = zeros
        return c
    lax.fori_loop(0, PAD // NL, _pad_head, 0)

    def _zero_hist(j, c):
        hist[pl.ds(NL * j, NL)] = zeros
        return c
    lax.fori_loop(0, HSIZE // NL, _zero_hist, 0, unroll=4)

    def _digit(ku, shift):
        t = ku ^ (lax.shift_right_arithmetic(ku, 31) | jnp.int32(INT_MIN))
        return lax.shift_right_logical(t, shift) & (RADIX - 1)

    def _row_body(r, carry):
        row = wid * ROWS_PER_W + r
        pltpu.sync_copy(xi_hbm.at[row], key_a.at[pl.ds(0, IN_SIZE)])

        def _pad_keys(j, c):
            key_a[pl.ds(IN_SIZE + NL * j, NL)] = zeros
            return c
        lax.fori_loop(0, PAD // NL, _pad_keys, 0)

        for p in range(4):
            if p % 2 == 0:
                src_k, src_v, dst_k, dst_v = key_a, val_a, key_b, val_b
            else:
                src_k, src_v, dst_k, dst_v = key_b, val_b, key_a, val_a
            shift = NBITS * p

            def _tail_mask(i):
                return (lane < NL - 1) | (i < LAST_LEN)

            def _a_load(i):
                hads = []
                for s in range(NSTREAM):
                    m = None if s < NSTREAM - 1 else _tail_mask(i)
                    ku = plsc.load_gather(src_k,
                                          [seg_base + (i + (NL * SEG) * s)],
                                          mask=m)
                    hads.append(_digit(ku, shift) * VL + (lane + NL * s))
                return tuple(hads)

            def _hist_body(i, hads):
                nxt = _a_load(jnp.minimum(i + 1, SEG - 1))
                for s in range(NSTREAM):
                    m = None if s < NSTREAM - 1 else _tail_mask(i)
                    plsc.addupdate_scatter(hist, [hads[s]], ones, mask=m)
                return nxt
            lax.fori_loop(0, SEG, _hist_body, _a_load(0))

            lane_is_last = lane == NL - 1
            def _b1_body(j, c):
                vs = [hist[pl.ds(NL * (4 * j + q), NL)] for q in range(4)]
                cs = [plsc.cumsum(v) for v in vs]
                for q in range(4):
                    hist[pl.ds(NL * (4 * j + q), NL)] = cs[q]
                    plsc.store_scatter(
                        tot, [jnp.full((NL,), 4 * j + q, jnp.int32)],
                        cs[q], mask=lane_is_last)
                return c
            lax.fori_loop(0, HSIZE // NL // 4, _b1_body, 0)

            def _b2_body(k, run):
                t = tot[pl.ds(NL * k, NL)]
                inc = plsc.cumsum(t)
                aux[pl.ds(NL * k, NL)] = (run + inc) - t
                return run + jnp.sum(t)
            lax.fori_loop(0, HSIZE // NL // NL, _b2_body, jnp.int32(0))

            lm1 = jnp.maximum(lane - 1, 0)
            def _b3_body(j, c):
                av = aux[pl.ds(4 * j, NL)]
                bases = [av[q] for q in range(4)]
                shs = [plsc.load_gather(hist,
                                        [NL * (4 * j + q) + lm1])
                       for q in range(4)]
                for q in range(4):
                    excl = jnp.where(lane > 0, shs[q], 0) + bases[q]
                    ctr[pl.ds(NL * (4 * j + q), NL)] = excl
                for q in range(4):
                    hist[pl.ds(NL * (4 * j + q), NL)] = zeros
                return c
            lax.fori_loop(0, HSIZE // NL // 4, _b3_body, 0)

            def _c_load(i):
                idxs = [seg_base + (i + (NL * SEG) * s)
                        for s in range(NSTREAM)]
                masks = [None if s < NSTREAM - 1 else _tail_mask(i)
                         for s in range(NSTREAM)]
                kus = [plsc.load_gather(src_k, [idxs[s]], mask=masks[s])
                       for s in range(NSTREAM)]
                if p == 0:
                    vvs = idxs
                else:
                    vvs = [plsc.load_gather(src_v, [idxs[s]], mask=masks[s])
                           for s in range(NSTREAM)]
                hads = [_digit(kus[s], shift) * VL + (lane + NL * s)
                        for s in range(NSTREAM)]
                return tuple(kus), tuple(vvs), tuple(hads)

            def _perm_body(i, st):
                kus, vvs, hads = st
                nxt = _c_load(jnp.minimum(i + 1, SEG - 1))
                dests = [plsc.load_gather(ctr, [hads[s]])
                         for s in range(NSTREAM)]
                for s in range(NSTREAM):
                    m = None if s < NSTREAM - 1 else _tail_mask(i)
                    plsc.store_scatter(ctr, [hads[s]], dests[s] + 1, mask=m)
                    plsc.store_scatter(dst_k, [dests[s]], kus[s], mask=m)
                    plsc.store_scatter(dst_v, [dests[s]], vvs[s], mask=m)
                return nxt
            lax.fori_loop(0, SEG, _perm_body, _c_load(0))

        def _f_load(j):
            svs = [val_a[pl.ds(NL * (NSTREAM * j + q), NL)]
                   for q in range(NSTREAM)]
            hvs = [plsc.load_gather(head_v, [svs[q]])
                   for q in range(NSTREAM)]
            return tuple(svs), tuple(hvs)

        def _final_body(j, st):
            svs, hvs = st
            nxt = _f_load(jnp.minimum(j + 1, NCH_OUT // NSTREAM - 1))
            for q in range(NSTREAM):
                key_b[pl.ds(NL * (NSTREAM * j + q), NL)] = hvs[q]
                plsc.store_scatter(val_b, [svs[q]],
                                   NL * (NSTREAM * j + q) + lane)
            return nxt
        lax.fori_loop(0, NCH_OUT // NSTREAM, _final_body, _f_load(0))

        pltpu.sync_copy(key_b, out_hbm.at[row])
        pltpu.sync_copy(val_b, rev_hbm.at[row])
        return carry

    lax.fori_loop(0, ROWS_PER_W, _row_body, 0)


@jax.jit
def kernel(x):
    xr = jnp.reshape(x, (ROWS, IN_SIZE))
    xi = lax.bitcast_convert_type(xr, jnp.int32)

    sc = pl.kernel(
        _sc_body,
        out_type=[
            jax.ShapeDtypeStruct((ROWS, OUT_SIZE), jnp.int32),
            jax.ShapeDtypeStruct((ROWS, OUT_SIZE), jnp.int32),
        ],
        mesh=plsc.VectorSubcoreMesh(core_axis_name="c", subcore_axis_name="s"),
        compiler_params=pltpu.CompilerParams(needs_layout_passes=False),
        scratch_types=[
            pltpu.VMEM((OUT_SIZE,), jnp.int32),
            pltpu.VMEM((OUT_SIZE,), jnp.int32),
            pltpu.VMEM((OUT_SIZE,), jnp.int32),
            pltpu.VMEM((OUT_SIZE,), jnp.int32),
            pltpu.VMEM((HSIZE,), jnp.int32),
            pltpu.VMEM((HSIZE,), jnp.int32),
            pltpu.VMEM((HSIZE // NL + NL,), jnp.int32),
            pltpu.VMEM((HSIZE // NL,), jnp.int32),
            pltpu.VMEM((OUT_SIZE,), jnp.int32),
        ],
    )
    out_bits, rev = sc(xi)
    out = lax.bitcast_convert_type(out_bits, jnp.float32)
    out = jnp.reshape(out, (B, C, 128, 130))
    rev = jnp.reshape(rev, (B, C, OUT_SIZE))
    return (out, rev)


# speedup vs baseline: 16.7734x; 1.0082x over previous
import functools

import jax
import jax.numpy as jnp
from jax import lax
from jax.experimental import pallas as pl
from jax.experimental.pallas import tpu as pltpu
from jax.experimental.pallas import tpu_sc as plsc

B, C = 64, 16
IN_SIZE = 128 * 128
OUT_SIZE = 128 * 130
PAD = OUT_SIZE - IN_SIZE
ROWS = B * C
NL = 16
NSTREAM = 4
VL = NL * NSTREAM
SEG = 261
LAST_LEN = OUT_SIZE - (VL - 1) * SEG
NCH_OUT = OUT_SIZE // NL
NC, NS = 2, 16
NW = NC * NS
ROWS_PER_W = ROWS // NW
NBITS = 8
RADIX = 1 << NBITS
HSIZE = RADIX * VL
INT_MIN = -2147483648


def _sc_body(xi_hbm, out_hbm, rev_hbm,
             key_a, key_b, val_a, val_b, stage, hist, aux, tot, head_v,
             in_sem, outk_sem, outv_sem):
    cid = lax.axis_index("c")
    sid = lax.axis_index("s")
    wid = sid * NC + cid

    lane = jnp.arange(NL, dtype=jnp.int32)
    seg_base = lane * SEG
    ones = jnp.ones((NL,), jnp.int32)
    zeros = jnp.zeros((NL,), jnp.int32)
    lane_is_last = lane == NL - 1

    pltpu.sync_copy(xi_hbm.at[0], head_v.at[pl.ds(0, IN_SIZE)])

    def _pad_init(j, c):
        head_v[pl.ds(IN_SIZE + NL * j, NL)] = zeros
        stage[pl.ds(IN_SIZE + NL * j, NL)] = zeros
        return c
    lax.fori_loop(0, PAD // NL, _pad_init, 0)

    row0 = wid * ROWS_PER_W
    pltpu.async_copy(xi_hbm.at[row0], stage.at[pl.ds(0, IN_SIZE)], in_sem)

    def _digit(ku, shift):
        t = ku ^ (lax.shift_right_arithmetic(ku, 31) | jnp.int32(INT_MIN))
        return lax.shift_right_logical(t, shift) & (RADIX - 1)

    def _tail_mask(i):
        return (lane < NL - 1) | (i < LAST_LEN)

    def _row_body(r, carry):
        row = wid * ROWS_PER_W + r
        pltpu.make_async_copy(xi_hbm.at[row], stage.at[pl.ds(0, IN_SIZE)],
                              in_sem).wait()

        for p in range(4):
            if p == 0:
                src_k, src_v, dst_k, dst_v = stage, None, key_b, val_b
            elif p == 1:
                src_k, src_v, dst_k, dst_v = key_b, val_b, key_a, val_a
            elif p == 2:
                src_k, src_v, dst_k, dst_v = key_a, val_a, key_b, val_b
            else:
                src_k, src_v, dst_k, dst_v = key_b, val_b, key_a, val_a
            shift = NBITS * p

            def _zero_hist(j, c):
                for q in range(4):
                    hist[pl.ds(NL * (4 * j + q), NL)] = zeros
                return c
            lax.fori_loop(0, HSIZE // NL // 4, _zero_hist, 0)

            def _a_load(i):
                hads = []
                for s in range(NSTREAM):
                    m = None if s < NSTREAM - 1 else _tail_mask(i)
                    ku = plsc.load_gather(src_k,
                                          [seg_base + (i + (NL * SEG) * s)],
                                          mask=m)
                    hads.append(_digit(ku, shift) * VL + (lane + NL * s))
                return tuple(hads)

            def _hist_body(i, hads):
                nxt = _a_load(jnp.minimum(i + 1, SEG - 1))
                for s in range(NSTREAM):
                    m = None if s < NSTREAM - 1 else _tail_mask(i)
                    plsc.addupdate_scatter(hist, [hads[s]], ones, mask=m)
                return nxt
            lax.fori_loop(0, SEG, _hist_body, _a_load(0))

            def _b1_body(j, c):
                vs = [hist[pl.ds(NL * (4 * j + q), NL)] for q in range(4)]
                cs = [plsc.cumsum(v) for v in vs]
                for q in range(4):
                    hist[pl.ds(NL * (4 * j + q), NL)] = cs[q]
                    plsc.store_scatter(
                        tot, [jnp.full((NL,), 4 * j + q, jnp.int32)],
                        cs[q], mask=lane_is_last)
                return c
            lax.fori_loop(0, HSIZE // NL // 4, _b1_body, 0)

            def _b2_body(k, run):
                t = tot[pl.ds(NL * k, NL)]
                inc = plsc.cumsum(t)
                aux[pl.ds(NL * k, NL)] = (run + inc) - t
                return run + jnp.sum(t)
            lax.fori_loop(0, HSIZE // NL // NL, _b2_body, jnp.int32(0))

            lm1 = jnp.maximum(lane - 1, 0)
            def _b3_body(j, c):
                av = aux[pl.ds(4 * j, NL)]
                shs = [plsc.load_gather(hist, [NL * (4 * j + q) + lm1])
                       for q in range(4)]
                for q in range(4):
                    excl = jnp.where(lane > 0, shs[q], 0) + av[q]
                    hist[pl.ds(NL * (4 * j + q), NL)] = excl
                return c
            lax.fori_loop(0, HSIZE // NL // 4, _b3_body, 0)

            if p == 0:
                @pl.when(r > 0)
                def _drain_out():
                    pltpu.make_async_copy(key_b, out_hbm.at[row - 1],
                                          outk_sem).wait()
                    pltpu.make_async_copy(val_b, rev_hbm.at[row - 1],
                                          outv_sem).wait()

            def _c_load(i):
                idxs = [seg_base + (i + (NL * SEG) * s)
                        for s in range(NSTREAM)]
                masks = [None if s < NSTREAM - 1 else _tail_mask(i)
                         for s in range(NSTREAM)]
                kus = [plsc.load_gather(src_k, [idxs[s]], mask=masks[s])
                       for s in range(NSTREAM)]
                if p == 0:
                    vvs = idxs
                else:
                    vvs = [plsc.load_gather(src_v, [idxs[s]],
                                            mask=masks[s])
                           for s in range(NSTREAM)]
                hads = [_digit(kus[s], shift) * VL + (lane + NL * s)
                        for s in range(NSTREAM)]
                return tuple(kus), tuple(vvs), tuple(hads)

            def _perm_body(i, st):
                kus, vvs, hads = st
                nxt = _c_load(jnp.minimum(i + 1, SEG - 1))
                dests = [plsc.load_gather(hist, [hads[s]])
                         for s in range(NSTREAM)]
                for s in range(NSTREAM):
                    m = None if s < NSTREAM - 1 else _tail_mask(i)
                    plsc.store_scatter(hist, [hads[s]], dests[s] + 1,
                                       mask=m)
                    plsc.store_scatter(dst_k, [dests[s]], kus[s], mask=m)
                    plsc.store_scatter(dst_v, [dests[s]], vvs[s], mask=m)
                return nxt
            lax.fori_loop(0, SEG, _perm_body, _c_load(0))

            if p == 0:
                @pl.when(r < ROWS_PER_W - 1)
                def _prefetch_next():
                    pltpu.async_copy(xi_hbm.at[row + 1],
                                     stage.at[pl.ds(0, IN_SIZE)], in_sem)

        def _f_load(j):
            svs = [val_a[pl.ds(NL * (NSTREAM * j + q), NL)]
                   for q in range(NSTREAM)]
            hvs = [plsc.load_gather(head_v, [svs[q]])
                   for q in range(NSTREAM)]
            return tuple(svs), tuple(hvs)

        def _final_body(j, st):
            svs, hvs = st
            nxt = _f_load(jnp.minimum(j + 1, NCH_OUT // NSTREAM - 1))
            for q in range(NSTREAM):
                key_b[pl.ds(NL * (NSTREAM * j + q), NL)] = hvs[q]
                plsc.store_scatter(val_b, [svs[q]],
                                   NL * (NSTREAM * j + q) + lane)
            return nxt
        lax.fori_loop(0, NCH_OUT // NSTREAM, _final_body, _f_load(0))

        pltpu.async_copy(key_b, out_hbm.at[row], outk_sem)
        pltpu.async_copy(val_b, rev_hbm.at[row], outv_sem)
        return carry

    lax.fori_loop(0, ROWS_PER_W, _row_body, 0)

    last = wid * ROWS_PER_W + ROWS_PER_W - 1
    pltpu.make_async_copy(key_b, out_hbm.at[last], outk_sem).wait()
    pltpu.make_async_copy(val_b, rev_hbm.at[last], outv_sem).wait()


@jax.jit
def kernel(x):
    xr = jnp.reshape(x, (ROWS, IN_SIZE))
    xi = lax.bitcast_convert_type(xr, jnp.int32)

    sc = pl.kernel(
        _sc_body,
        out_type=[
            jax.ShapeDtypeStruct((ROWS, OUT_SIZE), jnp.int32),
            jax.ShapeDtypeStruct((ROWS, OUT_SIZE), jnp.int32),
        ],
        mesh=plsc.VectorSubcoreMesh(core_axis_name="c", subcore_axis_name="s"),
        compiler_params=pltpu.CompilerParams(needs_layout_passes=False),
        scratch_types=[
            pltpu.VMEM((OUT_SIZE,), jnp.int32),
            pltpu.VMEM((OUT_SIZE,), jnp.int32),
            pltpu.VMEM((OUT_SIZE,), jnp.int32),
            pltpu.VMEM((OUT_SIZE,), jnp.int32),
            pltpu.VMEM((OUT_SIZE,), jnp.int32),
            pltpu.VMEM((HSIZE,), jnp.int32),
            pltpu.VMEM((HSIZE // NL + NL,), jnp.int32),
            pltpu.VMEM((HSIZE // NL,), jnp.int32),
            pltpu.VMEM((OUT_SIZE,), jnp.int32),
            pltpu.SemaphoreType.DMA,
            pltpu.SemaphoreType.DMA,
            pltpu.SemaphoreType.DMA,
        ],
    )
    out_bits, rev = sc(xi)
    out = lax.bitcast_convert_type(out_bits, jnp.float32)
    out = jnp.reshape(out, (B, C, 128, 130))
    rev = jnp.reshape(rev, (B, C, OUT_SIZE))
    return (out, rev)


# speedup vs baseline: 17.6670x; 1.0533x over previous
import functools

import jax
import jax.numpy as jnp
from jax import lax
from jax.experimental import pallas as pl
from jax.experimental.pallas import tpu as pltpu
from jax.experimental.pallas import tpu_sc as plsc

B, C = 64, 16
IN_SIZE = 128 * 128
OUT_SIZE = 128 * 130
PAD = OUT_SIZE - IN_SIZE
ROWS = B * C
NL = 16
NSTREAM = 4
VL = NL * NSTREAM
SEG = 261
LAST_LEN = OUT_SIZE - (VL - 1) * SEG
NCH_OUT = OUT_SIZE // NL
NC, NS = 2, 16
NW = NC * NS
ROWS_PER_W = ROWS // NW
NBITS = 8
RADIX = 1 << NBITS
HSIZE = RADIX * VL
INT_MIN = -2147483648


def _sc_body(xi_hbm, out_hbm, rev_hbm,
             key_a, key_b, val_a, val_b, stage, hist, aux, tot, head_v,
             in_sem, outk_sem, outv_sem):
    cid = lax.axis_index("c")
    sid = lax.axis_index("s")
    wid = sid * NC + cid

    lane = jnp.arange(NL, dtype=jnp.int32)
    seg_base = lane * SEG
    ones = jnp.ones((NL,), jnp.int32)
    zeros = jnp.zeros((NL,), jnp.int32)
    lane_is_last = lane == NL - 1

    pltpu.sync_copy(xi_hbm.at[0], head_v.at[pl.ds(0, IN_SIZE)])

    def _pad_init(j, c):
        head_v[pl.ds(IN_SIZE + NL * j, NL)] = zeros
        stage[pl.ds(IN_SIZE + NL * j, NL)] = zeros
        return c
    lax.fori_loop(0, PAD // NL, _pad_init, 0)

    row0 = wid * ROWS_PER_W
    pltpu.async_copy(xi_hbm.at[row0], stage.at[pl.ds(0, IN_SIZE)], in_sem)

    def _digit6(ku, shift):
        if shift >= 6:
            return lax.shift_right_logical(ku, shift - 6) & ((RADIX - 1) << 6)
        return lax.shift_left(ku, 6 - shift) & ((RADIX - 1) << 6)

    def _tail_mask(i):
        return (lane < NL - 1) | (i < LAST_LEN)

    def _xform(v):
        return v ^ (lax.shift_right_arithmetic(v, 31) | jnp.int32(INT_MIN))

    def _row_body(r, carry):
        row = wid * ROWS_PER_W + r
        pltpu.make_async_copy(xi_hbm.at[row], stage.at[pl.ds(0, IN_SIZE)],
                              in_sem).wait()

        for p in range(4):
            if p == 0:
                src_k, src_v, dst_k, dst_v = stage, None, key_b, val_b
            elif p == 1:
                src_k, src_v, dst_k, dst_v = key_b, val_b, key_a, val_a
            elif p == 2:
                src_k, src_v, dst_k, dst_v = key_a, val_a, key_b, val_b
            else:
                src_k, src_v, dst_k, dst_v = key_b, val_b, key_a, val_a
            shift = NBITS * p

            def _zero_hist(j, c):
                for q in range(4):
                    hist[pl.ds(NL * (4 * j + q), NL)] = zeros
                return c
            lax.fori_loop(0, HSIZE // NL // 4, _zero_hist, 0)

            def _a_load(i):
                hads = []
                for s in range(NSTREAM):
                    m = None if s < NSTREAM - 1 else _tail_mask(i)
                    ku = plsc.load_gather(src_k,
                                          [seg_base + (i + (NL * SEG) * s)],
                                          mask=m)
                    t = _xform(ku) if p == 0 else ku
                    hads.append(_digit6(t, shift) + (lane + NL * s))
                return tuple(hads)

            def _hist_body(i, hads):
                nxt = _a_load(jnp.minimum(i + 1, SEG - 1))
                for s in range(NSTREAM):
                    m = None if s < NSTREAM - 1 else _tail_mask(i)
                    plsc.addupdate_scatter(hist, [hads[s]], ones, mask=m)
                return nxt
            lax.fori_loop(0, SEG, _hist_body, _a_load(0))

            def _b1_body(j, c):
                vs = [hist[pl.ds(NL * (4 * j + q), NL)] for q in range(4)]
                cs = [plsc.cumsum(v) for v in vs]
                for q in range(4):
                    hist[pl.ds(NL * (4 * j + q), NL)] = cs[q]
                    plsc.store_scatter(
                        tot, [jnp.full((NL,), 4 * j + q, jnp.int32)],
                        cs[q], mask=lane_is_last)
                return c
            lax.fori_loop(0, HSIZE // NL // 4, _b1_body, 0)

            def _b2_body(k, run):
                t = tot[pl.ds(NL * k, NL)]
                inc = plsc.cumsum(t)
                aux[pl.ds(NL * k, NL)] = (run + inc) - t
                return run + jnp.sum(t)
            lax.fori_loop(0, HSIZE // NL // NL, _b2_body, jnp.int32(0))

            lm1 = jnp.maximum(lane - 1, 0)
            def _b3_body(j, c):
                av = aux[pl.ds(4 * j, NL)]
                shs = [plsc.load_gather(hist, [NL * (4 * j + q) + lm1])
                       for q in range(4)]
                for q in range(4):
                    excl = jnp.where(lane > 0, shs[q], 0) + av[q]
                    hist[pl.ds(NL * (4 * j + q), NL)] = excl
                return c
            lax.fori_loop(0, HSIZE // NL // 4, _b3_body, 0)

            if p == 1:
                @pl.when(r > 0)
                def _drain_out():
                    pltpu.make_async_copy(key_a, out_hbm.at[row - 1],
                                          outk_sem).wait()
                    pltpu.make_async_copy(val_a, rev_hbm.at[row - 1],
                                          outv_sem).wait()

            def _c_load(i):
                idxs = [seg_base + (i + (NL * SEG) * s)
                        for s in range(NSTREAM)]
                masks = [None if s < NSTREAM - 1 else _tail_mask(i)
                         for s in range(NSTREAM)]
                kus = [plsc.load_gather(src_k, [idxs[s]], mask=masks[s])
                       for s in range(NSTREAM)]
                if p == 0:
                    kus = [_xform(ku) for ku in kus]
                    vvs = idxs
                else:
                    vvs = [plsc.load_gather(src_v, [idxs[s]],
                                            mask=masks[s])
                           for s in range(NSTREAM)]
                hads = [_digit6(kus[s], shift) + (lane + NL * s)
                        for s in range(NSTREAM)]
                if p == 3:
                    hvs = [plsc.load_gather(head_v, [vvs[s]],
                                            mask=masks[s])
                           for s in range(NSTREAM)]
                else:
                    hvs = kus
                return tuple(kus), tuple(vvs), tuple(hads), tuple(hvs)

            def _perm_body(i, st):
                kus, vvs, hads, hvs = st
                nxt = _c_load(jnp.minimum(i + 1, SEG - 1))
                dests = [plsc.load_gather(hist, [hads[s]])
                         for s in range(NSTREAM)]
                for s in range(NSTREAM):
                    m = None if s < NSTREAM - 1 else _tail_mask(i)
                    plsc.store_scatter(hist, [hads[s]], dests[s] + 1,
                                       mask=m)
                    if p == 3:
                        plsc.store_scatter(dst_k, [dests[s]], hvs[s],
                                           mask=m)
                        plsc.store_scatter(dst_v, [vvs[s]], dests[s],
                                           mask=m)
                    else:
                        plsc.store_scatter(dst_k, [dests[s]], kus[s],
                                           mask=m)
                        plsc.store_scatter(dst_v, [dests[s]], vvs[s],
                                           mask=m)
                return nxt
            lax.fori_loop(0, SEG, _perm_body, _c_load(0))

            if p == 0:
                @pl.when(r < ROWS_PER_W - 1)
                def _prefetch_next():
                    pltpu.async_copy(xi_hbm.at[row + 1],
                                     stage.at[pl.ds(0, IN_SIZE)], in_sem)

        pltpu.async_copy(key_a, out_hbm.at[row], outk_sem)
        pltpu.async_copy(val_a, rev_hbm.at[row], outv_sem)
        return carry

    lax.fori_loop(0, ROWS_PER_W, _row_body, 0)

    last = wid * ROWS_PER_W + ROWS_PER_W - 1
    pltpu.make_async_copy(key_a, out_hbm.at[last], outk_sem).wait()
    pltpu.make_async_copy(val_a, rev_hbm.at[last], outv_sem).wait()


@jax.jit
def kernel(x):
    xr = jnp.reshape(x, (ROWS, IN_SIZE))
    xi = lax.bitcast_convert_type(xr, jnp.int32)

    sc = pl.kernel(
        _sc_body,
        out_type=[
            jax.ShapeDtypeStruct((ROWS, OUT_SIZE), jnp.int32),
            jax.ShapeDtypeStruct((ROWS, OUT_SIZE), jnp.int32),
        ],
        mesh=plsc.VectorSubcoreMesh(core_axis_name="c", subcore_axis_name="s"),
        compiler_params=pltpu.CompilerParams(needs_layout_passes=False),
        scratch_types=[
            pltpu.VMEM((OUT_SIZE,), jnp.int32),
            pltpu.VMEM((OUT_SIZE,), jnp.int32),
            pltpu.VMEM((OUT_SIZE,), jnp.int32),
            pltpu.VMEM((OUT_SIZE,), jnp.int32),
            pltpu.VMEM((OUT_SIZE,), jnp.int32),
            pltpu.VMEM((HSIZE,), jnp.int32),
            pltpu.VMEM((HSIZE // NL + NL,), jnp.int32),
            pltpu.VMEM((HSIZE // NL,), jnp.int32),
            pltpu.VMEM((OUT_SIZE,), jnp.int32),
            pltpu.SemaphoreType.DMA,
            pltpu.SemaphoreType.DMA,
            pltpu.SemaphoreType.DMA,
        ],
    )
    out_bits, rev = sc(xi)
    out = lax.bitcast_convert_type(out_bits, jnp.float32)
    out = jnp.reshape(out, (B, C, 128, 130))
    rev = jnp.reshape(rev, (B, C, OUT_SIZE))
    return (out, rev)
